# baseline (speedup 1.0000x reference)
import functools

import jax
import jax.numpy as jnp
from jax import lax
from jax.experimental import pallas as pl
from jax.experimental.pallas import tpu as pltpu
from jax.experimental.pallas import tpu_sc as plsc

BATCH = 16384
EMB = 32
NBLK = 1000001
NDIR = 1002
_LANES = 128
_TAIL0 = (NBLK // _LANES) * _LANES
_NTAIL = NBLK - _TAIL0

_info = plsc.get_sparse_core_info()
_NC = _info.num_cores
_NS = _info.num_subcores
_NP = _NS
_BPP = BATCH // _NP
_L = 16
_HF = EMB // 2
_NSLOT = 16


@functools.partial(
    pl.kernel,
    mesh=plsc.VectorSubcoreMesh(core_axis_name="c", subcore_axis_name="s"),
    out_type=jax.ShapeDtypeStruct((2 * EMB, BATCH), jnp.float32),
    compiler_params=pltpu.CompilerParams(needs_layout_passes=False),
    scratch_types=[
        pltpu.VMEM((_BPP,), jnp.int32),
        pltpu.VMEM((_BPP,), jnp.int32),
        pltpu.VMEM((_NSLOT, _HF, _LANES), jnp.float32),
        pltpu.VMEM((_HF, NDIR), jnp.float32),
        pltpu.VMEM((_NTAIL, EMB), jnp.float32),
        pltpu.VMEM((2 * _HF, _BPP), jnp.float32),
        pltpu.SemaphoreType.DMA,
        pltpu.SemaphoreType.DMA,
    ],
)
def _action_encoder(blk_idx_hbm, dir_idx_hbm, blk_t_hbm, dir_t_hbm,
                    tail_hbm, out_hbm, bidx_v, didx_v, strips_v, dtab_v,
                    tail_v, obuf_v, gsem, dsem):
    pair = lax.axis_index("s")
    h = lax.axis_index("c")
    base = pair * _BPP
    frow = h * _HF

    pltpu.sync_copy(blk_idx_hbm.at[pl.ds(base, _BPP)], bidx_v)
    pltpu.sync_copy(dir_idx_hbm.at[pl.ds(base, _BPP)], didx_v)
    dcopies = [pltpu.async_copy(tail_hbm, tail_v, dsem),
               pltpu.async_copy(dir_t_hbm.at[pl.ds(frow, _HF)], dtab_v,
                                dsem)]

    f0 = lax.iota(jnp.int32, _L)
    zeros = jnp.zeros((_L,), jnp.int32)

    def batch(t, _):
        k0 = t * _L
        iv = jnp.minimum(bidx_v[pl.ds(k0, _L)], _TAIL0 - 1)
        copies = []
        for s in range(_NSLOT):
            i = iv[s]
            col = pl.multiple_of((i >> 7) * _LANES, _LANES)
            copies.append(pltpu.async_copy(
                blk_t_hbm.at[pl.ds(frow, _HF), pl.ds(col, _LANES)],
                strips_v.at[s], gsem))
        for c in copies:
            c.wait()
        for s in range(_NSLOT):
            lv = zeros + (iv[s] & (_LANES - 1))
            kv = zeros + (k0 + s)
            sv = zeros + s
            g = plsc.load_gather(strips_v, [sv, f0, lv])
            plsc.store_scatter(obuf_v, [f0, kv], g)
        return _
    lax.fori_loop(0, _BPP // _L, batch, None)

    for c in dcopies:
        c.wait()

    def dir_chunk(c, _):
        di = didx_v[pl.ds(c * _L, _L)]
        for f in range(_HF):
            fv = jnp.zeros((_L,), jnp.int32) + f
            obuf_v[_HF + f, pl.ds(c * _L, _L)] = plsc.load_gather(
                dtab_v, [fv, di])
        return _
    lax.fori_loop(0, _BPP // _L, dir_chunk, None)

    def fixup(c, _):
        kv = lax.iota(jnp.int32, _L) + c * _L
        bv = bidx_v[pl.ds(c * _L, _L)]
        wv = bv - _TAIL0
        m = wv >= 0
        wc = jnp.maximum(wv, 0)
        for f in range(_HF):
            fv = jnp.zeros((_L,), jnp.int32) + f
            vals = plsc.load_gather(tail_v, [wc, fv + frow])
            plsc.store_scatter(obuf_v, [fv, kv], vals, mask=m)
        return _
    lax.fori_loop(0, _BPP // _L, fixup, None)

    pltpu.sync_copy(obuf_v.at[pl.ds(0, _HF)],
                    out_hbm.at[pl.ds(frow, _HF), pl.ds(base, _BPP)])
    pltpu.sync_copy(obuf_v.at[pl.ds(_HF, _HF)],
                    out_hbm.at[pl.ds(EMB + frow, _HF), pl.ds(base, _BPP)])


def kernel(block, direction, block_table, direction_table):
    blk = block.reshape(BATCH).astype(jnp.int32)
    dire = direction.reshape(BATCH).astype(jnp.int32)
    tail = block_table[_TAIL0:]
    out_t = _action_encoder(blk, dire, block_table.T, direction_table.T,
                            tail)
    return out_t.T


# speedup vs baseline: 1.2058x; 1.2058x over previous
import functools

import jax
import jax.numpy as jnp
from jax import lax
from jax.experimental import pallas as pl
from jax.experimental.pallas import tpu as pltpu
from jax.experimental.pallas import tpu_sc as plsc

BATCH = 16384
EMB = 32
NBLK = 1000001
NDIR = 1002
_LANES = 128
_TAIL0 = (NBLK // _LANES) * _LANES
_NTAIL = NBLK - _TAIL0

_info = plsc.get_sparse_core_info()
_NC = _info.num_cores
_NS = _info.num_subcores
_NP = _NS
_BPP = BATCH // _NP
_L = 16
_HF = EMB // 2
_NSLOT = 16


@functools.partial(
    pl.kernel,
    mesh=plsc.VectorSubcoreMesh(core_axis_name="c", subcore_axis_name="s"),
    out_type=jax.ShapeDtypeStruct((2 * EMB, BATCH), jnp.float32),
    compiler_params=pltpu.CompilerParams(needs_layout_passes=False),
    scratch_types=[
        pltpu.VMEM((_BPP,), jnp.int32),
        pltpu.VMEM((_BPP,), jnp.int32),
        pltpu.VMEM((2 * _NSLOT, _HF, _LANES), jnp.float32),
        pltpu.VMEM((_HF, NDIR), jnp.float32),
        pltpu.VMEM((_NTAIL, EMB), jnp.float32),
        pltpu.VMEM((2 * _HF, _BPP), jnp.float32),
        pltpu.SemaphoreType.DMA,
        pltpu.SemaphoreType.DMA,
        pltpu.SemaphoreType.DMA,
    ],
)
def _action_encoder(blk_idx_hbm, dir_idx_hbm, blk_t_hbm, dir_t_hbm,
                    tail_hbm, out_hbm, bidx_v, didx_v, strips_v, dtab_v,
                    tail_v, obuf_v, gsem, dsem, dsem2):
    pair = lax.axis_index("s")
    h = lax.axis_index("c")
    base = pair * _BPP
    frow = h * _HF

    pltpu.sync_copy(blk_idx_hbm.at[pl.ds(base, _BPP)], bidx_v)
    pltpu.sync_copy(dir_idx_hbm.at[pl.ds(base, _BPP)], didx_v)
    dcopies = [pltpu.async_copy(tail_hbm, tail_v, dsem),
               pltpu.async_copy(dir_t_hbm.at[pl.ds(frow, _HF)], dtab_v,
                                dsem)]

    f0 = lax.iota(jnp.int32, _L)
    zeros = jnp.zeros((_L,), jnp.int32)
    ngrp = _BPP // _L

    def fire(g, slot0, sem):
        iv = jnp.minimum(bidx_v[pl.ds(g * _L, _L)], _TAIL0 - 1)
        for s in range(_NSLOT):
            col = pl.multiple_of((iv[s] >> 7) * _LANES, _LANES)
            pltpu.async_copy(
                blk_t_hbm.at[pl.ds(frow, _HF), pl.ds(col, _LANES)],
                strips_v.at[slot0 + s], sem)

    def drain(sem):
        fake = pltpu.make_async_copy(
            blk_t_hbm.at[pl.ds(0, _HF), pl.ds(0, _LANES)],
            strips_v.at[0], sem)
        for _ in range(_NSLOT):
            fake.wait()

    def extract(g, slot0):
        k0 = g * _L
        iv = jnp.minimum(bidx_v[pl.ds(k0, _L)], _TAIL0 - 1)
        for s in range(_NSLOT):
            lv = zeros + (iv[s] & (_LANES - 1))
            kv = zeros + (k0 + s)
            sv = zeros + (slot0 + s)
            g0 = plsc.load_gather(strips_v, [sv, f0, lv])
            plsc.store_scatter(obuf_v, [f0, kv], g0)

    fire(0, 0, gsem)

    def batch(tt, _):
        ga = 2 * tt
        fire(ga + 1, _NSLOT, dsem2)
        drain(gsem)
        extract(ga, 0)
        fire(jnp.minimum(ga + 2, ngrp - 1), 0, gsem)
        drain(dsem2)
        extract(ga + 1, _NSLOT)
        return _
    lax.fori_loop(0, ngrp // 2, batch, None)
    drain(gsem)

    for c in dcopies:
        c.wait()

    def dir_chunk(c, _):
        di = didx_v[pl.ds(c * _L, _L)]
        for f in range(_HF):
            fv = jnp.zeros((_L,), jnp.int32) + f
            obuf_v[_HF + f, pl.ds(c * _L, _L)] = plsc.load_gather(
                dtab_v, [fv, di])
        return _
    lax.fori_loop(0, _BPP // _L, dir_chunk, None)

    def fixup(c, _):
        kv = lax.iota(jnp.int32, _L) + c * _L
        bv = bidx_v[pl.ds(c * _L, _L)]
        wv = bv - _TAIL0
        m = wv >= 0
        wc = jnp.maximum(wv, 0)
        for f in range(_HF):
            fv = jnp.zeros((_L,), jnp.int32) + f
            vals = plsc.load_gather(tail_v, [wc, fv + frow])
            plsc.store_scatter(obuf_v, [fv, kv], vals, mask=m)
        return _
    lax.fori_loop(0, _BPP // _L, fixup, None)

    pltpu.sync_copy(obuf_v.at[pl.ds(0, _HF)],
                    out_hbm.at[pl.ds(frow, _HF), pl.ds(base, _BPP)])
    pltpu.sync_copy(obuf_v.at[pl.ds(_HF, _HF)],
                    out_hbm.at[pl.ds(EMB + frow, _HF), pl.ds(base, _BPP)])


def kernel(block, direction, block_table, direction_table):
    blk = block.reshape(BATCH).astype(jnp.int32)
    dire = direction.reshape(BATCH).astype(jnp.int32)
    tail = block_table[_TAIL0:]
    out_t = _action_encoder(blk, dire, block_table.T, direction_table.T,
                            tail)
    return out_t.T
